# baseline (speedup 1.0000x reference)
import jax
import jax.numpy as jnp
from jax import lax
from jax.experimental import pallas as pl
from jax.experimental.pallas import tpu as pltpu
from jax.experimental.pallas import tpu_sc as plsc

N, D, T, C = 50000, 256, 10000, 8
NC, NS = 2, 16
NW = NC * NS
BPW = 320
LASTW = NW - 1
LAST_BASE = LASTW * BPW
LAST_N = T - LAST_BASE

RB = 10000
NBLK = N // RB
TB = 5000
NTB = T // TB
NOUT = N + T + C


def _sc_gather_body(feat_hbm, lab_hbm, idx_hbm, rows_out, labct_out,
                    idx_v, rows_v, labct_v, sem_r, sem_l):
    wid = lax.axis_index("s") * NC + lax.axis_index("c")
    base = wid * BPW

    def chunk(n):
        pltpu.sync_copy(idx_hbm.at[pl.ds(base, n)], idx_v.at[pl.ds(0, n)])
        rows_dma = pltpu.async_copy(
            feat_hbm.at[idx_v.at[pl.ds(0, n)]], rows_v.at[pl.ds(0, n)], sem_r)
        lab_dma = pltpu.async_copy(
            lab_hbm.at[idx_v.at[pl.ds(0, n)]], labct_v.at[pl.ds(0, n)], sem_l)
        lab_dma.wait()
        pltpu.sync_copy(labct_v.at[pl.ds(0, n)],
                        labct_out.at[pl.ds(base, n)])
        rows_dma.wait()
        pltpu.sync_copy(rows_v.at[pl.ds(0, n)],
                        rows_out.at[pl.ds(base, n)])

    @pl.when(wid < LASTW)
    def _full():
        chunk(BPW)

    @pl.when(wid == LASTW)
    def _tail():
        chunk(LAST_N)


def _sc_gather(features, labels, chosen_tails):
    mesh = plsc.VectorSubcoreMesh(core_axis_name="c", subcore_axis_name="s")
    return pl.kernel(
        _sc_gather_body,
        out_type=(
            jax.ShapeDtypeStruct((T, D), jnp.float32),
            jax.ShapeDtypeStruct((T,), jnp.int32),
        ),
        mesh=mesh,
        scratch_types=[
            pltpu.VMEM((BPW,), jnp.int32),
            pltpu.VMEM((BPW, D), jnp.float32),
            pltpu.VMEM((BPW,), jnp.int32),
            pltpu.SemaphoreType.DMA,
            pltpu.SemaphoreType.DMA,
        ],
    )(features, labels, chosen_tails)


def _segsum_body(feat_ref, lab_ref, out_ref, sums_ref, counts_ref):
    i = pl.program_id(0)

    @pl.when(i == 0)
    def _init():
        sums_ref[...] = jnp.zeros_like(sums_ref)
        counts_ref[...] = jnp.zeros_like(counts_ref)

    out_ref[...] = feat_ref[...]
    lab = lab_ref[0]
    cls = lax.broadcasted_iota(jnp.int32, (16, RB), 0) + 1
    oh = (lab == cls).astype(jnp.float32)
    sums_ref[...] += jax.lax.dot_general(
        oh, feat_ref[...], (((1,), (0,)), ((), ())),
        preferred_element_type=jnp.float32)
    cnt = jnp.sum(oh, axis=1, keepdims=True)
    counts_ref[...] += jnp.broadcast_to(cnt, counts_ref.shape)


def _segsum(features, labels3d):
    return pl.pallas_call(
        _segsum_body,
        grid=(NBLK,),
        in_specs=[
            pl.BlockSpec((RB, D), lambda i: (i, 0)),
            pl.BlockSpec((1, 1, RB), lambda i: (i, 0, 0)),
        ],
        out_specs=[
            pl.BlockSpec((RB, D), lambda i: (i, 0)),
            pl.BlockSpec((16, D), lambda i: (0, 0)),
            pl.BlockSpec((16, 128), lambda i: (0, 0)),
        ],
        out_shape=[
            jax.ShapeDtypeStruct((NOUT, D), jnp.float32),
            jax.ShapeDtypeStruct((16, D), jnp.float32),
            jax.ShapeDtypeStruct((16, 128), jnp.float32),
        ],
    )(features, labels3d)


def _interp_body(big_ref, tails_ref, labct_ref, sums_ref, counts_ref, w_ref,
                 out_ref):
    del big_ref
    j = pl.program_id(0)
    counts = counts_ref[:, 0:1]

    @pl.when(j < NTB)
    def _interp():
        safe = sums_ref[...] * (1.0 / jnp.maximum(counts, 1.0))
        lab = labct_ref[0]
        cls = lax.broadcasted_iota(jnp.int32, (16, TB), 0) + 1
        oh = (lab == cls).astype(jnp.float32)
        ce = jax.lax.dot_general(
            oh, safe, (((0,), (0,)), ((), ())),
            preferred_element_type=jnp.float32)
        tails = tails_ref[...]
        out_ref[...] = tails + jax.lax.dot_general(
            tails - ce, w_ref[...], (((1,), (0,)), ((), ())),
            preferred_element_type=jnp.float32)

    @pl.when(j == NTB)
    def _centers():
        out_ref[0:8, :] = sums_ref[0:8, :] / counts[0:8]


def _interp(big, tails, labct3d, sums, counts, w):
    return pl.pallas_call(
        _interp_body,
        grid=(NTB + 1,),
        in_specs=[
            pl.BlockSpec(memory_space=pl.ANY),
            pl.BlockSpec((TB, D), lambda j: (jnp.minimum(j, NTB - 1), 0)),
            pl.BlockSpec((1, 1, TB), lambda j: (jnp.minimum(j, NTB - 1), 0, 0)),
            pl.BlockSpec((16, D), lambda j: (0, 0)),
            pl.BlockSpec((16, 128), lambda j: (0, 0)),
            pl.BlockSpec((D, D), lambda j: (0, 0)),
        ],
        out_specs=pl.BlockSpec((TB, D), lambda j: (N // TB + j, 0)),
        out_shape=jax.ShapeDtypeStruct((NOUT, D), jnp.float32),
        input_output_aliases={0: 0},
    )(big, tails, labct3d, sums, counts, w)


def kernel(features, labels, chosen_tails, sm_weight_center):
    big, sums, counts = _segsum(features, labels.reshape(NBLK, 1, RB))
    tails, labct = _sc_gather(features, labels, chosen_tails)
    return _interp(big, tails, labct.reshape(NTB, 1, TB),
                   sums, counts, sm_weight_center)


# speedup vs baseline: 1.0161x; 1.0161x over previous
import jax
import jax.numpy as jnp
from jax import lax
from jax.experimental import pallas as pl
from jax.experimental.pallas import tpu as pltpu
from jax.experimental.pallas import tpu_sc as plsc

N, D, T, C = 50000, 256, 10000, 8
NC, NS = 2, 16
NW = NC * NS
BPW = 320
LASTW = NW - 1
LAST_BASE = LASTW * BPW
LAST_N = T - LAST_BASE

RB = 25000
NBLK = N // RB
TB = 5000
NTB = T // TB
NOUT = N + T + C


def _sc_gather_body(feat_hbm, lab_hbm, idx_hbm, rows_out, labct_out,
                    idx_v, rows_v, labct_v, sem_r, sem_l):
    wid = lax.axis_index("s") * NC + lax.axis_index("c")
    base = wid * BPW

    def chunk(n):
        pltpu.sync_copy(idx_hbm.at[pl.ds(base, n)], idx_v.at[pl.ds(0, n)])
        rows_dma = pltpu.async_copy(
            feat_hbm.at[idx_v.at[pl.ds(0, n)]], rows_v.at[pl.ds(0, n)], sem_r)
        lab_dma = pltpu.async_copy(
            lab_hbm.at[idx_v.at[pl.ds(0, n)]], labct_v.at[pl.ds(0, n)], sem_l)
        lab_dma.wait()
        pltpu.sync_copy(labct_v.at[pl.ds(0, n)],
                        labct_out.at[pl.ds(base, n)])
        rows_dma.wait()
        pltpu.sync_copy(rows_v.at[pl.ds(0, n)],
                        rows_out.at[pl.ds(base, n)])

    @pl.when(wid < LASTW)
    def _full():
        chunk(BPW)

    @pl.when(wid == LASTW)
    def _tail():
        chunk(LAST_N)


def _sc_gather(features, labels, chosen_tails):
    mesh = plsc.VectorSubcoreMesh(core_axis_name="c", subcore_axis_name="s")
    return pl.kernel(
        _sc_gather_body,
        out_type=(
            jax.ShapeDtypeStruct((T, D), jnp.float32),
            jax.ShapeDtypeStruct((T,), jnp.int32),
        ),
        mesh=mesh,
        scratch_types=[
            pltpu.VMEM((BPW,), jnp.int32),
            pltpu.VMEM((BPW, D), jnp.float32),
            pltpu.VMEM((BPW,), jnp.int32),
            pltpu.SemaphoreType.DMA,
            pltpu.SemaphoreType.DMA,
        ],
    )(features, labels, chosen_tails)


def _segsum_body(feat_ref, lab_ref, out_ref, sums_ref, counts_ref):
    i = pl.program_id(0)
    j = pl.program_id(1)

    @pl.when(jnp.logical_and(i == 0, j == 0))
    def _init():
        sums_ref[...] = jnp.zeros_like(sums_ref)
        counts_ref[...] = jnp.zeros_like(counts_ref)

    out_ref[...] = feat_ref[...]
    lab = lab_ref[0]
    cls = lax.broadcasted_iota(jnp.int32, (16, RB), 0) + 1
    oh = (lab == cls).astype(jnp.float32)
    sums_ref[...] += jax.lax.dot_general(
        oh, feat_ref[...], (((1,), (0,)), ((), ())),
        preferred_element_type=jnp.float32)

    @pl.when(j == 0)
    def _cnt():
        cnt = jnp.sum(oh, axis=1, keepdims=True)
        counts_ref[...] += jnp.broadcast_to(cnt, counts_ref.shape)


def _segsum(features, labels3d):
    return pl.pallas_call(
        _segsum_body,
        grid=(NBLK, 2),
        in_specs=[
            pl.BlockSpec((RB, D // 2), lambda i, j: (i, j)),
            pl.BlockSpec((1, 1, RB), lambda i, j: (i, 0, 0)),
        ],
        out_specs=[
            pl.BlockSpec((RB, D // 2), lambda i, j: (i, j)),
            pl.BlockSpec((16, D // 2), lambda i, j: (0, j)),
            pl.BlockSpec((16, 128), lambda i, j: (0, 0)),
        ],
        out_shape=[
            jax.ShapeDtypeStruct((NOUT, D), jnp.float32),
            jax.ShapeDtypeStruct((16, D), jnp.float32),
            jax.ShapeDtypeStruct((16, 128), jnp.float32),
        ],
    )(features, labels3d)


def _interp_body(big_ref, tails_ref, labct_ref, sums_ref, counts_ref, w_ref,
                 out_ref):
    del big_ref
    j = pl.program_id(0)
    counts = counts_ref[:, 0:1]

    @pl.when(j < NTB)
    def _interp():
        safe = sums_ref[...] * (1.0 / jnp.maximum(counts, 1.0))
        lab = labct_ref[0]
        cls = lax.broadcasted_iota(jnp.int32, (16, TB), 0) + 1
        oh = (lab == cls).astype(jnp.float32)
        ce = jax.lax.dot_general(
            oh, safe, (((0,), (0,)), ((), ())),
            preferred_element_type=jnp.float32)
        tails = tails_ref[...]
        out_ref[...] = tails + jax.lax.dot_general(
            tails - ce, w_ref[...], (((1,), (0,)), ((), ())),
            preferred_element_type=jnp.float32)

    @pl.when(j == NTB)
    def _centers():
        out_ref[0:8, :] = sums_ref[0:8, :] / counts[0:8]


def _interp(big, tails, labct3d, sums, counts, w):
    return pl.pallas_call(
        _interp_body,
        grid=(NTB + 1,),
        in_specs=[
            pl.BlockSpec(memory_space=pl.ANY),
            pl.BlockSpec((TB, D), lambda j: (jnp.minimum(j, NTB - 1), 0)),
            pl.BlockSpec((1, 1, TB), lambda j: (jnp.minimum(j, NTB - 1), 0, 0)),
            pl.BlockSpec((16, D), lambda j: (0, 0)),
            pl.BlockSpec((16, 128), lambda j: (0, 0)),
            pl.BlockSpec((D, D), lambda j: (0, 0)),
        ],
        out_specs=pl.BlockSpec((TB, D), lambda j: (N // TB + j, 0)),
        out_shape=jax.ShapeDtypeStruct((NOUT, D), jnp.float32),
        input_output_aliases={0: 0},
    )(big, tails, labct3d, sums, counts, w)


def kernel(features, labels, chosen_tails, sm_weight_center):
    big, sums, counts = _segsum(features, labels.reshape(NBLK, 1, RB))
    tails, labct = _sc_gather(features, labels, chosen_tails)
    return _interp(big, tails, labct.reshape(NTB, 1, TB),
                   sums, counts, sm_weight_center)


# speedup vs baseline: 1.0303x; 1.0140x over previous
import jax
import jax.numpy as jnp
from jax import lax
from jax.experimental import pallas as pl
from jax.experimental.pallas import tpu as pltpu
from jax.experimental.pallas import tpu_sc as plsc

N, D, T, C = 50000, 256, 10000, 8
NC, NS = 2, 16
NW = NC * NS
BPW = 320
LASTW = NW - 1
LAST_BASE = LASTW * BPW
LAST_N = T - LAST_BASE

RB = 25000
NBLK = N // RB
TB = 5000
NTB = T // TB
NOUT = N + T + C


def _sc_gather_body(feat_hbm, lab_hbm, idx_hbm, rows_out, labct_out,
                    idx_v, rows_v, labct_v, sem_r, sem_l):
    wid = lax.axis_index("s") * NC + lax.axis_index("c")
    base = wid * BPW

    def chunk(n):
        pltpu.sync_copy(idx_hbm.at[pl.ds(base, n)], idx_v.at[pl.ds(0, n)])
        rows_dma = pltpu.async_copy(
            feat_hbm.at[idx_v.at[pl.ds(0, n)]], rows_v.at[pl.ds(0, n)], sem_r)
        lab_dma = pltpu.async_copy(
            lab_hbm.at[idx_v.at[pl.ds(0, n)]], labct_v.at[pl.ds(0, n)], sem_l)
        lab_dma.wait()
        pltpu.sync_copy(labct_v.at[pl.ds(0, n)],
                        labct_out.at[pl.ds(base, n)])
        rows_dma.wait()
        pltpu.sync_copy(rows_v.at[pl.ds(0, n)],
                        rows_out.at[pl.ds(base, n)])

    @pl.when(wid < LASTW)
    def _full():
        chunk(BPW)

    @pl.when(wid == LASTW)
    def _tail():
        chunk(LAST_N)


def _sc_gather(features, labels, chosen_tails):
    mesh = plsc.VectorSubcoreMesh(core_axis_name="c", subcore_axis_name="s")
    return pl.kernel(
        _sc_gather_body,
        out_type=(
            jax.ShapeDtypeStruct((T, D), jnp.float32),
            jax.ShapeDtypeStruct((T,), jnp.int32),
        ),
        mesh=mesh,
        scratch_types=[
            pltpu.VMEM((BPW,), jnp.int32),
            pltpu.VMEM((BPW, D), jnp.float32),
            pltpu.VMEM((BPW,), jnp.int32),
            pltpu.SemaphoreType.DMA,
            pltpu.SemaphoreType.DMA,
        ],
    )(features, labels, chosen_tails)


def _segsum_body(feat_ref, lab_ref, out_ref, sums_ref, counts_ref):
    i = pl.program_id(0)
    j = pl.program_id(1)

    @pl.when(jnp.logical_and(i == 0, j == 0))
    def _init():
        sums_ref[...] = jnp.zeros_like(sums_ref)
        counts_ref[...] = jnp.zeros_like(counts_ref)

    out_ref[...] = feat_ref[...]
    lab = lab_ref[0]
    cls = lax.broadcasted_iota(jnp.int32, (16, RB), 0) + 1
    oh = (lab == cls).astype(jnp.float32)
    sums_ref[...] += jax.lax.dot_general(
        oh, feat_ref[...], (((1,), (0,)), ((), ())),
        preferred_element_type=jnp.float32)

    @pl.when(j == 0)
    def _cnt():
        cnt = jnp.sum(oh, axis=1, keepdims=True)
        counts_ref[...] += jnp.broadcast_to(cnt, counts_ref.shape)


def _segsum(features, labels3d):
    return pl.pallas_call(
        _segsum_body,
        grid=(NBLK, 2),
        in_specs=[
            pl.BlockSpec((RB, D // 2), lambda i, j: (i, j)),
            pl.BlockSpec((1, 1, RB), lambda i, j: (i, 0, 0)),
        ],
        out_specs=[
            pl.BlockSpec((RB, D // 2), lambda i, j: (i, j)),
            pl.BlockSpec((16, D // 2), lambda i, j: (0, j)),
            pl.BlockSpec((16, 128), lambda i, j: (0, 0)),
        ],
        out_shape=[
            jax.ShapeDtypeStruct((NOUT, D), jnp.float32),
            jax.ShapeDtypeStruct((16, D), jnp.float32),
            jax.ShapeDtypeStruct((16, 128), jnp.float32),
        ],
    )(features, labels3d)


def _interp_body(big_ref, tails_ref, labct_ref, sums_ref, counts_ref, w_ref,
                 out_ref):
    del big_ref
    j = pl.program_id(0)
    counts = counts_ref[:, 0:1]

    @pl.when(j < NTB)
    def _interp():
        safe = sums_ref[...] * (1.0 / jnp.maximum(counts, 1.0))
        lab = labct_ref[0]
        cls = lax.broadcasted_iota(jnp.int32, (16, TB), 0) + 1
        oh = (lab == cls).astype(jnp.float32)
        ce = jax.lax.dot_general(
            oh, safe, (((0,), (0,)), ((), ())),
            preferred_element_type=jnp.float32)
        tails = tails_ref[...]
        out_ref[...] = tails + jax.lax.dot_general(
            tails - ce, w_ref[...], (((1,), (0,)), ((), ())),
            preferred_element_type=jnp.float32)

    @pl.when(j == NTB)
    def _centers():
        out_ref[0:8, :] = sums_ref[0:8, :] / counts[0:8]


def _interp(big, tails, labct3d, sums, counts, w):
    return pl.pallas_call(
        _interp_body,
        grid=(NTB + 1,),
        in_specs=[
            pl.BlockSpec(memory_space=pl.ANY),
            pl.BlockSpec((TB, D), lambda j: (jnp.minimum(j, NTB - 1), 0)),
            pl.BlockSpec((1, 1, TB), lambda j: (jnp.minimum(j, NTB - 1), 0, 0)),
            pl.BlockSpec((16, D), lambda j: (0, 0)),
            pl.BlockSpec((16, 128), lambda j: (0, 0)),
            pl.BlockSpec((D, D), lambda j: (0, 0)),
        ],
        out_specs=pl.BlockSpec((TB, D), lambda j: (N // TB + j, 0)),
        out_shape=jax.ShapeDtypeStruct((NOUT, D), jnp.float32),
        input_output_aliases={0: 0},
    )(big, tails, labct3d, sums, counts, w)


def kernel(features, labels, chosen_tails, sm_weight_center):
    labels3d = labels.reshape(NBLK, 1, RB)
    labels3d, chosen_tails = lax.optimization_barrier((labels3d, chosen_tails))
    big, sums, counts = _segsum(features, labels3d)
    tails, labct = _sc_gather(features, labels, chosen_tails)
    return _interp(big, tails, labct.reshape(NTB, 1, TB),
                   sums, counts, sm_weight_center)
